# baseline (speedup 1.0000x reference)
import functools

import jax
import jax.numpy as jnp
from jax import lax
from jax.experimental import pallas as pl
from jax.experimental.pallas import tpu as pltpu
from jax.experimental.pallas import tpu_sc as plsc

N = 10000
E = 320000
F_IN = 128
HID = 128
NH = 8
CH = 16
NL = 4
NG = 64

WID = 144
NC = 2
NS = 16
NW = NC * NS
EPW = E // NW
BE = 40
NB = EPW // BE
NSL = 4
NP = 10240
RPT = NP // NS
ZR = 128
LAST = N - 15 * RPT

RB = 1000
f32 = jnp.float32
i32 = jnp.int32


def _sc_body(xa_hbm, adst_hbm, src_hbm, dst_hbm,
             outp_hbm,
             out_sh, *bufs):
    cid = lax.axis_index("c")
    sid = lax.axis_index("s")
    wid = sid * NC + cid
    zero16 = jnp.zeros((16,), f32)
    xa_b = bufs[0:NSL]
    ad_b = bufs[NSL:2 * NSL]
    si = bufs[2 * NSL:3 * NSL]
    di = bufs[3 * NSL:4 * NSL]
    sg = bufs[4 * NSL:5 * NSL]
    ss = bufs[5 * NSL:6 * NSL]
    sv = bufs[6 * NSL:7 * NSL]

    @pl.loop(0, BE)
    def _(r):
        for c in range(WID // 16):
            xa_b[0][r, pl.ds(c * 16, 16)] = zero16

    r0 = sid * RPT
    for k in range(RPT // BE):
        pltpu.sync_copy(xa_b[0], out_sh.at[pl.ds(r0 + k * BE, BE)])

    hvecs = [jnp.full((16,), h, dtype=i32) for h in range(NH)]
    base_w = wid * EPW

    def idx_start(j, m):
        b0 = base_w + j * BE
        pltpu.async_copy(src_hbm.at[pl.ds(b0, BE)], si[m], sv[m])
        pltpu.async_copy(dst_hbm.at[pl.ds(b0, BE)], di[m], sv[m])

    def idx_wait(j, m):
        b0 = base_w + j * BE
        pltpu.make_async_copy(src_hbm.at[pl.ds(b0, BE)], si[m], sv[m]).wait()
        pltpu.make_async_copy(dst_hbm.at[pl.ds(b0, BE)], di[m], sv[m]).wait()

    def gather_start(m):
        pltpu.async_copy(xa_hbm.at[si[m]], xa_b[m], sg[m])
        pltpu.async_copy(adst_hbm.at[di[m]], ad_b[m], sg[m])

    def gather_wait(m):
        pltpu.make_async_copy(xa_hbm.at[si[m]], xa_b[m], sg[m]).wait()
        pltpu.make_async_copy(adst_hbm.at[di[m]], ad_b[m], sg[m]).wait()

    def scatter_start(m):
        pltpu.async_copy(xa_b[m], out_sh.at[di[m]], ss[m], add=True)

    def scatter_wait(m):
        pltpu.make_async_copy(xa_b[m], out_sh.at[di[m]], ss[m]).wait()

    def compute(m):
        xap, adp = xa_b[m], ad_b[m]
        sl_a = pl.ds(HID, 16)

        @plsc.parallel_loop(0, BE, unroll=2)
        def _(b):
            s = xap[b, sl_a] + adp[b, :]
            s = jnp.maximum(s, s * 0.2)
            w = jnp.exp(s)
            for h in range(NH):
                wh = w.at[hvecs[h]].get(mode="promise_in_bounds")
                sl = pl.ds(h * 16, 16)
                xap[b, sl] = xap[b, sl] * wh
            xap[b, sl_a] = w

    idx_start(0, 0)
    idx_wait(0, 0)
    gather_start(0)
    idx_start(1, 1)
    plsc.subcore_barrier()

    @pl.loop(0, (NB + NSL - 1) // NSL)
    def _(jj):
        for r in range(NSL):
            j = jj * NSL + r
            m = r
            m1 = (r + 1) % NSL
            m2 = (r + 2) % NSL

            @pl.when(j < NB)
            def _():
                @pl.when(j >= 2)
                def _():
                    scatter_wait(m2)

                @pl.when(j + 1 < NB)
                def _():
                    idx_wait(j + 1, m1)
                    gather_start(m1)

                @pl.when(j + 2 < NB)
                def _():
                    idx_start(j + 2, m2)

                gather_wait(m)
                compute(m)
                scatter_start(m)

    scatter_wait((NB - 2) % NSL)
    scatter_wait((NB - 1) % NSL)

    plsc.subcore_barrier()

    @pl.when(sid < NS - 1)
    def _():
        pltpu.sync_copy(out_sh.at[pl.ds(r0, RPT)],
                        outp_hbm.at[cid, pl.ds(r0, RPT)])

    @pl.when(sid == NS - 1)
    def _():
        pltpu.sync_copy(out_sh.at[pl.ds((NS - 1) * RPT, LAST)],
                        outp_hbm.at[cid, pl.ds((NS - 1) * RPT, LAST)])


_sc_layer = pl.kernel(
    _sc_body,
    out_type=jax.ShapeDtypeStruct((NC, N, WID), f32),
    mesh=plsc.VectorSubcoreMesh(core_axis_name="c", subcore_axis_name="s"),
    compiler_params=pltpu.CompilerParams(needs_layout_passes=False,
                                         use_tc_tiling_on_sc=False),
    scratch_types=[
        pltpu.VMEM_SHARED((NP, WID), f32),
        *([pltpu.VMEM((BE, WID), f32)] * NSL),
        *([pltpu.VMEM((BE, CH), f32)] * NSL),
        *([pltpu.VMEM((BE,), i32)] * (2 * NSL)),
        *([pltpu.SemaphoreType.DMA] * (3 * NSL)),
    ],
)


def _dot(a, b):
    return jnp.dot(a, b, preferred_element_type=f32)


def _proj_body(x_ref, w_ref, b_ref, ws_ref, ssrc_ref, sdst_ref,
               xa_ref, ad_ref):
    h = jnp.maximum(_dot(x_ref[...], w_ref[...]) + b_ref[...], 0.0)
    xw = _dot(h, ws_ref[...])
    xa_ref[...] = jnp.concatenate([xw, _dot(xw, ssrc_ref[...])], axis=1)
    ad_ref[...] = _dot(xw, sdst_ref[...])


def _proj(x, W_in, b_in2, Ws0, Ssrc0, Sdst0):
    return pl.pallas_call(
        _proj_body,
        grid=(N // RB,),
        in_specs=[
            pl.BlockSpec((RB, F_IN), lambda i: (i, 0)),
            pl.BlockSpec((F_IN, HID), lambda i: (0, 0)),
            pl.BlockSpec((1, HID), lambda i: (0, 0)),
            pl.BlockSpec((HID, HID), lambda i: (0, 0)),
            pl.BlockSpec((HID, CH), lambda i: (0, 0)),
            pl.BlockSpec((HID, CH), lambda i: (0, 0)),
        ],
        out_specs=[
            pl.BlockSpec((RB, WID), lambda i: (i, 0)),
            pl.BlockSpec((RB, CH), lambda i: (i, 0)),
        ],
        out_shape=[
            jax.ShapeDtypeStruct((N, WID), f32),
            jax.ShapeDtypeStruct((N, CH), f32),
        ],
    )(x, W_in, b_in2, Ws0, Ssrc0, Sdst0)


def _layer_post(outp, xa, a_d, sc, off, rmat):
    tot144 = outp[0] + outp[1]
    xw = xa[:, :HID]
    a_s = xa[:, HID:]
    sv = a_s + a_d
    wself = jnp.exp(jnp.maximum(sv, sv * 0.2))
    tot = tot144[:, :HID] + _dot(wself, rmat) * xw
    esum = tot144[:, HID:] + wself
    recip = 1.0 / (esum + 1e-16)
    return jnp.maximum(tot * _dot(recip, rmat) * sc + off, 0.0)


def _combine_body(outp_ref, xa_ref, ad_ref,
                  sc_ref, off_ref, rmat_ref, ws_ref, ssrc_ref, sdst_ref,
                  xan_ref, adn_ref):
    hn = _layer_post(outp_ref[...], xa_ref[...], ad_ref[...],
                     sc_ref[...], off_ref[...], rmat_ref[...])
    xwn = _dot(hn, ws_ref[...])
    xan_ref[...] = jnp.concatenate([xwn, _dot(xwn, ssrc_ref[...])], axis=1)
    adn_ref[...] = _dot(xwn, sdst_ref[...])


def _combine(outp, xa, Ad, sc, off, rmat, Wsn, Ssrcn, Sdstn):
    return pl.pallas_call(
        _combine_body,
        grid=(N // RB,),
        in_specs=[
            pl.BlockSpec((NC, RB, WID), lambda i: (0, i, 0)),
            pl.BlockSpec((RB, WID), lambda i: (i, 0)),
            pl.BlockSpec((RB, CH), lambda i: (i, 0)),
            pl.BlockSpec((1, HID), lambda i: (0, 0)),
            pl.BlockSpec((1, HID), lambda i: (0, 0)),
            pl.BlockSpec((CH, HID), lambda i: (0, 0)),
            pl.BlockSpec((HID, HID), lambda i: (0, 0)),
            pl.BlockSpec((HID, CH), lambda i: (0, 0)),
            pl.BlockSpec((HID, CH), lambda i: (0, 0)),
        ],
        out_specs=[
            pl.BlockSpec((RB, WID), lambda i: (i, 0)),
            pl.BlockSpec((RB, CH), lambda i: (i, 0)),
        ],
        out_shape=[
            jax.ShapeDtypeStruct((N, WID), f32),
            jax.ShapeDtypeStruct((N, CH), f32),
        ],
    )(outp, xa, Ad, sc, off, rmat, Wsn, Ssrcn, Sdstn)


def _final_body(outp_ref, xa_ref, ad_ref,
                sc_ref, off_ref, rmat_ref, batch_ref,
                w1_ref, b1_ref, w2_ref, b2_ref, out_ref,
                pool_acc, cnt_acc):
    i = pl.program_id(0)

    @pl.when(i == 0)
    def _():
        pool_acc[...] = jnp.zeros_like(pool_acc)
        cnt_acc[...] = jnp.zeros_like(cnt_acc)

    hn = _layer_post(outp_ref[...], xa_ref[...], ad_ref[...],
                     sc_ref[...], off_ref[...], rmat_ref[...])
    b2d = jnp.reshape(batch_ref[...], (1, RB))
    onehot_t = (lax.broadcasted_iota(i32, (NG, RB), 0) == b2d).astype(f32)
    pool_acc[...] += lax.dot_general(onehot_t, hn, (((1,), (0,)), ((), ())),
                                     preferred_element_type=f32)
    cnt_acc[...] += lax.dot_general(onehot_t, jnp.ones((RB, 1), f32),
                                    (((1,), (0,)), ((), ())),
                                    preferred_element_type=f32)

    @pl.when(i == (N // RB) - 1)
    def _():
        pooled = pool_acc[...] / jnp.maximum(cnt_acc[...], 1.0)
        t = jnp.maximum(_dot(pooled, w1_ref[...]) + b1_ref[...], 0.0)
        out_ref[...] = _dot(t, w2_ref[...]) + b2_ref[...]


def _final(outp, xa, Ad, sc, off, rmat, batch3,
           W1cat, b1cat, W2bd, b2row):
    return pl.pallas_call(
        _final_body,
        grid=(N // RB,),
        in_specs=[
            pl.BlockSpec((NC, RB, WID), lambda i: (0, i, 0)),
            pl.BlockSpec((RB, WID), lambda i: (i, 0)),
            pl.BlockSpec((RB, CH), lambda i: (i, 0)),
            pl.BlockSpec((1, HID), lambda i: (0, 0)),
            pl.BlockSpec((1, HID), lambda i: (0, 0)),
            pl.BlockSpec((CH, HID), lambda i: (0, 0)),
            pl.BlockSpec((1, 1, RB), lambda i: (i, 0, 0)),
            pl.BlockSpec((HID, 3 * NG), lambda i: (0, 0)),
            pl.BlockSpec((1, 3 * NG), lambda i: (0, 0)),
            pl.BlockSpec((3 * NG, 3), lambda i: (0, 0)),
            pl.BlockSpec((1, 3), lambda i: (0, 0)),
        ],
        out_specs=pl.BlockSpec((NG, 3), lambda i: (0, 0)),
        out_shape=jax.ShapeDtypeStruct((NG, 3), f32),
        scratch_shapes=[
            pltpu.VMEM((NG, HID), f32),
            pltpu.VMEM((NG, 1), f32),
        ],
    )(outp, xa, Ad, sc, off, rmat, batch3,
      W1cat, b1cat, W2bd, b2row)


def kernel(x, edge_index, batch, W_in, b_in, Ws, att_src, att_dst, biases,
           bn_gamma, bn_beta, bn_mean, bn_var, HW1, Hb1, HW2, Hb2):
    src = edge_index[0]
    dst = edge_index[1]

    eye8 = jnp.eye(NH, dtype=f32)
    S_src = (att_src[:, :, :, None] * eye8[:, None, :][None]).reshape(
        NL, HID, NH)
    S_src = jnp.pad(S_src, ((0, 0), (0, 0), (0, CH - NH)))
    S_dst = (att_dst[:, :, :, None] * eye8[:, None, :][None]).reshape(
        NL, HID, NH)
    S_dst = jnp.pad(S_dst, ((0, 0), (0, 0), (0, CH - NH)))

    scale = bn_gamma / jnp.sqrt(bn_var + 1e-5)
    off2 = biases * scale + (bn_beta - bn_mean * scale)

    rmat = (jnp.arange(HID, dtype=i32)[None, :] // CH
            == jnp.arange(CH, dtype=i32)[:, None]).astype(f32)

    W1cat = jnp.concatenate([HW1[0], HW1[1], HW1[2]], axis=1)
    b1cat = jnp.concatenate([Hb1[0], Hb1[1], Hb1[2]])[None, :]
    W2bd = (HW2[:, :, 0][:, :, None] * jnp.eye(3, dtype=f32)[:, None, :]
            ).reshape(3 * NG, 3)
    b2row = Hb2[:, 0][None, :]

    batch3 = batch.reshape(N // RB, 1, RB)
    b_in2 = b_in[None, :]

    xa, Ad = _proj(x, W_in, b_in2, Ws[0], S_src[0], S_dst[0])
    for l in range(NL):
        outp = _sc_layer(xa, Ad, src, dst)
        if l + 1 < NL:
            xa, Ad = _combine(outp, xa, Ad,
                              scale[l][None], off2[l][None], rmat,
                              Ws[l + 1], S_src[l + 1], S_dst[l + 1])
        else:
            out = _final(outp, xa, Ad,
                         scale[l][None], off2[l][None], rmat, batch3,
                         W1cat, b1cat, W2bd, b2row)
    return out


# speedup vs baseline: 1.0799x; 1.0799x over previous
import functools

import jax
import jax.numpy as jnp
from jax import lax
from jax.experimental import pallas as pl
from jax.experimental.pallas import tpu as pltpu
from jax.experimental.pallas import tpu_sc as plsc

N = 10000
E = 320000
F_IN = 128
HID = 128
NH = 8
CH = 16
NL = 4
NG = 64

NC = 2
NS = 16
NW = NC * NS
EPW = E // NW
BE = 40
NB = EPW // BE
NSL = 4
NP = 10240
RPT = NP // NS
LAST = N - 15 * RPT

RB = 1000
f32 = jnp.float32
i32 = jnp.int32


def _sc_body(xw_hbm, asrc_hbm, adst_hbm, ei_hbm,
             outp_hbm, esump_hbm,
             out_sh, esum_sh, *bufs):
    cid = lax.axis_index("c")
    sid = lax.axis_index("s")
    wid = sid * NC + cid
    zero16 = jnp.zeros((16,), f32)
    xw_b = bufs[0:NSL]
    as_b = bufs[NSL:2 * NSL]
    ad_b = bufs[2 * NSL:3 * NSL]
    w_b = bufs[3 * NSL:4 * NSL]
    ei_b = bufs[4 * NSL:5 * NSL]
    sg = bufs[5 * NSL:6 * NSL]
    ss = bufs[6 * NSL:7 * NSL]
    sv = bufs[7 * NSL:8 * NSL]
    si = [e.at[0] for e in ei_b]
    di = [e.at[1] for e in ei_b]

    @pl.loop(0, BE)
    def _(r):
        for c in range(HID // 16):
            xw_b[0][r, pl.ds(c * 16, 16)] = zero16
        w_b[0][r, :] = zero16

    r0 = sid * RPT
    for k in range(RPT // BE):
        pltpu.sync_copy(xw_b[0], out_sh.at[pl.ds(r0 + k * BE, BE)])
        pltpu.sync_copy(w_b[0], esum_sh.at[pl.ds(r0 + k * BE, BE)])

    hvecs = [jnp.full((16,), h, dtype=i32) for h in range(NH)]
    base_w = wid * EPW

    def idx_start(j, m):
        b0 = base_w + j * BE
        pltpu.async_copy(ei_hbm.at[:, pl.ds(b0, BE)], ei_b[m], sv[m])

    def idx_wait(j, m):
        b0 = base_w + j * BE
        pltpu.make_async_copy(ei_hbm.at[:, pl.ds(b0, BE)], ei_b[m],
                              sv[m]).wait()

    def gather_start(m):
        pltpu.async_copy(xw_hbm.at[si[m]], xw_b[m], sg[m])
        pltpu.async_copy(asrc_hbm.at[si[m]], as_b[m], sg[m])
        pltpu.async_copy(adst_hbm.at[di[m]], ad_b[m], sg[m])

    def gather_wait(m):
        pltpu.make_async_copy(xw_hbm.at[si[m]], xw_b[m], sg[m]).wait()
        pltpu.make_async_copy(asrc_hbm.at[si[m]], as_b[m], sg[m]).wait()
        pltpu.make_async_copy(adst_hbm.at[di[m]], ad_b[m], sg[m]).wait()

    def scatter_start(m):
        pltpu.async_copy(xw_b[m], out_sh.at[di[m]], ss[m], add=True)
        pltpu.async_copy(w_b[m], esum_sh.at[di[m]], ss[m], add=True)

    def scatter_wait(m):
        pltpu.make_async_copy(xw_b[m], out_sh.at[di[m]], ss[m]).wait()
        pltpu.make_async_copy(w_b[m], esum_sh.at[di[m]], ss[m]).wait()

    def compute(m):
        xwp, asp, adp, wp = xw_b[m], as_b[m], ad_b[m], w_b[m]

        @plsc.parallel_loop(0, BE, unroll=4)
        def _(b):
            s = asp[b, :] + adp[b, :]
            s = jnp.maximum(s, s * 0.2)
            w = jnp.exp(s)
            wp[b, :] = w
            for h in range(NH):
                wh = w.at[hvecs[h]].get(mode="promise_in_bounds")
                sl = pl.ds(h * 16, 16)
                xwp[b, sl] = xwp[b, sl] * wh

    idx_start(0, 0)
    idx_wait(0, 0)
    gather_start(0)
    idx_start(1, 1)
    plsc.subcore_barrier()

    @pl.loop(0, (NB + NSL - 1) // NSL)
    def _(jj):
        for r in range(NSL):
            j = jj * NSL + r
            m = r
            m1 = (r + 1) % NSL
            m2 = (r + 2) % NSL

            @pl.when(j < NB)
            def _():
                @pl.when(j >= 2)
                def _():
                    scatter_wait(m2)

                @pl.when(j + 1 < NB)
                def _():
                    idx_wait(j + 1, m1)
                    gather_start(m1)

                @pl.when(j + 2 < NB)
                def _():
                    idx_start(j + 2, m2)

                gather_wait(m)
                compute(m)
                scatter_start(m)

    scatter_wait((NB - 2) % NSL)
    scatter_wait((NB - 1) % NSL)
    plsc.subcore_barrier()

    @pl.when(sid < NS - 1)
    def _():
        pltpu.sync_copy(out_sh.at[pl.ds(r0, RPT)],
                        outp_hbm.at[cid, pl.ds(r0, RPT)])
        pltpu.sync_copy(esum_sh.at[pl.ds(r0, RPT)],
                        esump_hbm.at[cid, pl.ds(r0, RPT)])

    @pl.when(sid == NS - 1)
    def _():
        pltpu.sync_copy(out_sh.at[pl.ds((NS - 1) * RPT, LAST)],
                        outp_hbm.at[cid, pl.ds((NS - 1) * RPT, LAST)])
        pltpu.sync_copy(esum_sh.at[pl.ds((NS - 1) * RPT, LAST)],
                        esump_hbm.at[cid, pl.ds((NS - 1) * RPT, LAST)])


_sc_layer = pl.kernel(
    _sc_body,
    out_type=(jax.ShapeDtypeStruct((NC, N, HID), f32),
              jax.ShapeDtypeStruct((NC, N, CH), f32)),
    mesh=plsc.VectorSubcoreMesh(core_axis_name="c", subcore_axis_name="s"),
    compiler_params=pltpu.CompilerParams(needs_layout_passes=False,
                                         use_tc_tiling_on_sc=False),
    scratch_types=[
        pltpu.VMEM_SHARED((NP, HID), f32),
        pltpu.VMEM_SHARED((NP, CH), f32),
        *([pltpu.VMEM((BE, HID), f32)] * NSL),
        *([pltpu.VMEM((BE, CH), f32)] * (3 * NSL)),
        *([pltpu.VMEM((2, BE), i32)] * NSL),
        *([pltpu.SemaphoreType.DMA] * (3 * NSL)),
    ],
)


def _dot(a, b):
    return jnp.dot(a, b, preferred_element_type=f32)


def _proj_body(x_ref, w_ref, b_ref, ws_ref, ssrc_ref, sdst_ref,
               xw_ref, as_ref, ad_ref):
    h = jnp.maximum(_dot(x_ref[...], w_ref[...]) + b_ref[...], 0.0)
    xw = _dot(h, ws_ref[...])
    xw_ref[...] = xw
    as_ref[...] = _dot(xw, ssrc_ref[...])
    ad_ref[...] = _dot(xw, sdst_ref[...])


def _proj(x, W_in, b_in2, Ws0, Ssrc0, Sdst0):
    return pl.pallas_call(
        _proj_body,
        grid=(N // RB,),
        in_specs=[
            pl.BlockSpec((RB, F_IN), lambda i: (i, 0)),
            pl.BlockSpec((F_IN, HID), lambda i: (0, 0)),
            pl.BlockSpec((1, HID), lambda i: (0, 0)),
            pl.BlockSpec((HID, HID), lambda i: (0, 0)),
            pl.BlockSpec((HID, CH), lambda i: (0, 0)),
            pl.BlockSpec((HID, CH), lambda i: (0, 0)),
        ],
        out_specs=[
            pl.BlockSpec((RB, HID), lambda i: (i, 0)),
            pl.BlockSpec((RB, CH), lambda i: (i, 0)),
            pl.BlockSpec((RB, CH), lambda i: (i, 0)),
        ],
        out_shape=[
            jax.ShapeDtypeStruct((N, HID), f32),
            jax.ShapeDtypeStruct((N, CH), f32),
            jax.ShapeDtypeStruct((N, CH), f32),
        ],
    )(x, W_in, b_in2, Ws0, Ssrc0, Sdst0)


def _layer_post(outp, esump, xw, a_s, a_d, sc, off, rmat):
    sv = a_s + a_d
    wself = jnp.exp(jnp.maximum(sv, sv * 0.2))
    tot = outp[0] + outp[1] + _dot(wself, rmat) * xw
    esum = esump[0] + esump[1] + wself
    recip = 1.0 / (esum + 1e-16)
    return jnp.maximum(tot * _dot(recip, rmat) * sc + off, 0.0)


def _combine_body(outp_ref, esump_ref, xw_ref, as_ref, ad_ref,
                  sc_ref, off_ref, rmat_ref, ws_ref, ssrc_ref, sdst_ref,
                  xwn_ref, asn_ref, adn_ref):
    hn = _layer_post(outp_ref[...], esump_ref[...], xw_ref[...],
                     as_ref[...], ad_ref[...],
                     sc_ref[...], off_ref[...], rmat_ref[...])
    xwn = _dot(hn, ws_ref[...])
    xwn_ref[...] = xwn
    asn_ref[...] = _dot(xwn, ssrc_ref[...])
    adn_ref[...] = _dot(xwn, sdst_ref[...])


def _combine(outp, esump, xw, As, Ad, sc, off, rmat, Wsn, Ssrcn, Sdstn):
    return pl.pallas_call(
        _combine_body,
        grid=(N // RB,),
        in_specs=[
            pl.BlockSpec((NC, RB, HID), lambda i: (0, i, 0)),
            pl.BlockSpec((NC, RB, CH), lambda i: (0, i, 0)),
            pl.BlockSpec((RB, HID), lambda i: (i, 0)),
            pl.BlockSpec((RB, CH), lambda i: (i, 0)),
            pl.BlockSpec((RB, CH), lambda i: (i, 0)),
            pl.BlockSpec((1, HID), lambda i: (0, 0)),
            pl.BlockSpec((1, HID), lambda i: (0, 0)),
            pl.BlockSpec((CH, HID), lambda i: (0, 0)),
            pl.BlockSpec((HID, HID), lambda i: (0, 0)),
            pl.BlockSpec((HID, CH), lambda i: (0, 0)),
            pl.BlockSpec((HID, CH), lambda i: (0, 0)),
        ],
        out_specs=[
            pl.BlockSpec((RB, HID), lambda i: (i, 0)),
            pl.BlockSpec((RB, CH), lambda i: (i, 0)),
            pl.BlockSpec((RB, CH), lambda i: (i, 0)),
        ],
        out_shape=[
            jax.ShapeDtypeStruct((N, HID), f32),
            jax.ShapeDtypeStruct((N, CH), f32),
            jax.ShapeDtypeStruct((N, CH), f32),
        ],
    )(outp, esump, xw, As, Ad, sc, off, rmat, Wsn, Ssrcn, Sdstn)


def _final_body(outp_ref, esump_ref, xw_ref, as_ref, ad_ref,
                sc_ref, off_ref, rmat_ref, batch_ref,
                w1_ref, b1_ref, w2_ref, b2_ref, out_ref,
                pool_acc, cnt_acc):
    i = pl.program_id(0)

    @pl.when(i == 0)
    def _():
        pool_acc[...] = jnp.zeros_like(pool_acc)
        cnt_acc[...] = jnp.zeros_like(cnt_acc)

    hn = _layer_post(outp_ref[...], esump_ref[...], xw_ref[...],
                     as_ref[...], ad_ref[...],
                     sc_ref[...], off_ref[...], rmat_ref[...])
    b2d = jnp.reshape(batch_ref[...], (1, RB))
    onehot_t = (lax.broadcasted_iota(i32, (NG, RB), 0) == b2d).astype(f32)
    pool_acc[...] += lax.dot_general(onehot_t, hn, (((1,), (0,)), ((), ())),
                                     preferred_element_type=f32)
    cnt_acc[...] += lax.dot_general(onehot_t, jnp.ones((RB, 1), f32),
                                    (((1,), (0,)), ((), ())),
                                    preferred_element_type=f32)

    @pl.when(i == (N // RB) - 1)
    def _():
        pooled = pool_acc[...] / jnp.maximum(cnt_acc[...], 1.0)
        t = jnp.maximum(_dot(pooled, w1_ref[...]) + b1_ref[...], 0.0)
        out_ref[...] = _dot(t, w2_ref[...]) + b2_ref[...]


def _final(outp, esump, xw, As, Ad, sc, off, rmat, batch3,
           W1cat, b1cat, W2bd, b2row):
    return pl.pallas_call(
        _final_body,
        grid=(N // RB,),
        in_specs=[
            pl.BlockSpec((NC, RB, HID), lambda i: (0, i, 0)),
            pl.BlockSpec((NC, RB, CH), lambda i: (0, i, 0)),
            pl.BlockSpec((RB, HID), lambda i: (i, 0)),
            pl.BlockSpec((RB, CH), lambda i: (i, 0)),
            pl.BlockSpec((RB, CH), lambda i: (i, 0)),
            pl.BlockSpec((1, HID), lambda i: (0, 0)),
            pl.BlockSpec((1, HID), lambda i: (0, 0)),
            pl.BlockSpec((CH, HID), lambda i: (0, 0)),
            pl.BlockSpec((1, 1, RB), lambda i: (i, 0, 0)),
            pl.BlockSpec((HID, 3 * NG), lambda i: (0, 0)),
            pl.BlockSpec((1, 3 * NG), lambda i: (0, 0)),
            pl.BlockSpec((3 * NG, 3), lambda i: (0, 0)),
            pl.BlockSpec((1, 3), lambda i: (0, 0)),
        ],
        out_specs=pl.BlockSpec((NG, 3), lambda i: (0, 0)),
        out_shape=jax.ShapeDtypeStruct((NG, 3), f32),
        scratch_shapes=[
            pltpu.VMEM((NG, HID), f32),
            pltpu.VMEM((NG, 1), f32),
        ],
    )(outp, esump, xw, As, Ad, sc, off, rmat, batch3,
      W1cat, b1cat, W2bd, b2row)


def kernel(x, edge_index, batch, W_in, b_in, Ws, att_src, att_dst, biases,
           bn_gamma, bn_beta, bn_mean, bn_var, HW1, Hb1, HW2, Hb2):
    eye8 = jnp.eye(NH, dtype=f32)
    S_src = (att_src[:, :, :, None] * eye8[:, None, :][None]).reshape(
        NL, HID, NH)
    S_src = jnp.pad(S_src, ((0, 0), (0, 0), (0, CH - NH)))
    S_dst = (att_dst[:, :, :, None] * eye8[:, None, :][None]).reshape(
        NL, HID, NH)
    S_dst = jnp.pad(S_dst, ((0, 0), (0, 0), (0, CH - NH)))

    scale = bn_gamma / jnp.sqrt(bn_var + 1e-5)
    off2 = biases * scale + (bn_beta - bn_mean * scale)

    rmat = (jnp.arange(HID, dtype=i32)[None, :] // CH
            == jnp.arange(CH, dtype=i32)[:, None]).astype(f32)

    W1cat = jnp.concatenate([HW1[0], HW1[1], HW1[2]], axis=1)
    b1cat = jnp.concatenate([Hb1[0], Hb1[1], Hb1[2]])[None, :]
    W2bd = (HW2[:, :, 0][:, :, None] * jnp.eye(3, dtype=f32)[:, None, :]
            ).reshape(3 * NG, 3)
    b2row = Hb2[:, 0][None, :]

    batch3 = batch.reshape(N // RB, 1, RB)
    b_in2 = b_in[None, :]

    xw, As, Ad = _proj(x, W_in, b_in2, Ws[0], S_src[0], S_dst[0])
    for l in range(NL):
        outp, esump = _sc_layer(xw, As, Ad, edge_index)
        if l + 1 < NL:
            xw, As, Ad = _combine(outp, esump, xw, As, Ad,
                                  scale[l][None], off2[l][None], rmat,
                                  Ws[l + 1], S_src[l + 1], S_dst[l + 1])
        else:
            out = _final(outp, esump, xw, As, Ad,
                         scale[l][None], off2[l][None], rmat, batch3,
                         W1cat, b1cat, W2bd, b2row)
    return out


# speedup vs baseline: 1.0806x; 1.0007x over previous
import functools

import jax
import jax.numpy as jnp
from jax import lax
from jax.experimental import pallas as pl
from jax.experimental.pallas import tpu as pltpu
from jax.experimental.pallas import tpu_sc as plsc

N = 10000
E = 320000
F_IN = 128
HID = 128
NH = 8
CH = 16
NL = 4
NG = 64

NC = 2
NS = 16
NW = NC * NS
EPW = E // NW
BE = 40
NB = EPW // BE
NSL = 5
NP = 10240
RPT = NP // NS
LAST = N - 15 * RPT

RB = 1000
f32 = jnp.float32
i32 = jnp.int32


def _sc_body(xw_hbm, asrc_hbm, adst_hbm, ei_hbm,
             outp_hbm, esump_hbm,
             out_sh, esum_sh, *bufs):
    cid = lax.axis_index("c")
    sid = lax.axis_index("s")
    wid = sid * NC + cid
    zero16 = jnp.zeros((16,), f32)
    xw_b = bufs[0:NSL]
    as_b = bufs[NSL:2 * NSL]
    ad_b = bufs[2 * NSL:3 * NSL]
    w_b = bufs[3 * NSL:4 * NSL]
    ei_b = bufs[4 * NSL:5 * NSL]
    sg = bufs[5 * NSL:6 * NSL]
    ss = bufs[6 * NSL:7 * NSL]
    sv = bufs[7 * NSL:8 * NSL]
    si = [e.at[0] for e in ei_b]
    di = [e.at[1] for e in ei_b]

    @pl.loop(0, BE)
    def _(r):
        for c in range(HID // 16):
            xw_b[0][r, pl.ds(c * 16, 16)] = zero16
        w_b[0][r, :] = zero16

    r0 = sid * RPT
    for k in range(RPT // BE):
        pltpu.sync_copy(xw_b[0], out_sh.at[pl.ds(r0 + k * BE, BE)])
        pltpu.sync_copy(w_b[0], esum_sh.at[pl.ds(r0 + k * BE, BE)])

    hvecs = [jnp.full((16,), h, dtype=i32) for h in range(NH)]
    base_w = wid * EPW

    def idx_start(j, m):
        b0 = base_w + j * BE
        pltpu.async_copy(ei_hbm.at[:, pl.ds(b0, BE)], ei_b[m], sv[m])

    def idx_wait(j, m):
        b0 = base_w + j * BE
        pltpu.make_async_copy(ei_hbm.at[:, pl.ds(b0, BE)], ei_b[m],
                              sv[m]).wait()

    def gather_start(m):
        pltpu.async_copy(xw_hbm.at[si[m]], xw_b[m], sg[m])
        pltpu.async_copy(asrc_hbm.at[si[m]], as_b[m], sg[m])
        pltpu.async_copy(adst_hbm.at[di[m]], ad_b[m], sg[m])

    def gather_wait(m):
        pltpu.make_async_copy(xw_hbm.at[si[m]], xw_b[m], sg[m]).wait()
        pltpu.make_async_copy(asrc_hbm.at[si[m]], as_b[m], sg[m]).wait()
        pltpu.make_async_copy(adst_hbm.at[di[m]], ad_b[m], sg[m]).wait()

    def scatter_start(m):
        pltpu.async_copy(xw_b[m], out_sh.at[di[m]], ss[m], add=True)
        pltpu.async_copy(w_b[m], esum_sh.at[di[m]], ss[m], add=True)

    def scatter_wait(m):
        pltpu.make_async_copy(xw_b[m], out_sh.at[di[m]], ss[m]).wait()
        pltpu.make_async_copy(w_b[m], esum_sh.at[di[m]], ss[m]).wait()

    def compute(m):
        xwp, asp, adp, wp = xw_b[m], as_b[m], ad_b[m], w_b[m]

        @plsc.parallel_loop(0, BE, unroll=4)
        def _(b):
            s = asp[b, :] + adp[b, :]
            s = jnp.maximum(s, s * 0.2)
            w = jnp.exp(s)
            wp[b, :] = w
            for h in range(NH):
                wh = w.at[hvecs[h]].get(mode="promise_in_bounds")
                sl = pl.ds(h * 16, 16)
                xwp[b, sl] = xwp[b, sl] * wh

    idx_start(0, 0)
    idx_wait(0, 0)
    gather_start(0)
    idx_start(1, 1)
    plsc.subcore_barrier()

    @pl.loop(0, (NB + NSL - 1) // NSL)
    def _(jj):
        for r in range(NSL):
            j = jj * NSL + r
            m = r
            m1 = (r + 1) % NSL
            m2 = (r + 2) % NSL

            @pl.when(j < NB)
            def _():
                @pl.when(j >= 3)
                def _():
                    scatter_wait(m2)

                @pl.when(j + 1 < NB)
                def _():
                    idx_wait(j + 1, m1)
                    gather_start(m1)

                @pl.when(j + 2 < NB)
                def _():
                    idx_start(j + 2, m2)

                gather_wait(m)
                compute(m)
                scatter_start(m)

    scatter_wait((NB - 3) % NSL)
    scatter_wait((NB - 2) % NSL)
    scatter_wait((NB - 1) % NSL)
    plsc.subcore_barrier()

    @pl.when(sid < NS - 1)
    def _():
        pltpu.sync_copy(out_sh.at[pl.ds(r0, RPT)],
                        outp_hbm.at[cid, pl.ds(r0, RPT)])
        pltpu.sync_copy(esum_sh.at[pl.ds(r0, RPT)],
                        esump_hbm.at[cid, pl.ds(r0, RPT)])

    @pl.when(sid == NS - 1)
    def _():
        pltpu.sync_copy(out_sh.at[pl.ds((NS - 1) * RPT, LAST)],
                        outp_hbm.at[cid, pl.ds((NS - 1) * RPT, LAST)])
        pltpu.sync_copy(esum_sh.at[pl.ds((NS - 1) * RPT, LAST)],
                        esump_hbm.at[cid, pl.ds((NS - 1) * RPT, LAST)])


_sc_layer = pl.kernel(
    _sc_body,
    out_type=(jax.ShapeDtypeStruct((NC, N, HID), f32),
              jax.ShapeDtypeStruct((NC, N, CH), f32)),
    mesh=plsc.VectorSubcoreMesh(core_axis_name="c", subcore_axis_name="s"),
    compiler_params=pltpu.CompilerParams(needs_layout_passes=False,
                                         use_tc_tiling_on_sc=False),
    scratch_types=[
        pltpu.VMEM_SHARED((NP, HID), f32),
        pltpu.VMEM_SHARED((NP, CH), f32),
        *([pltpu.VMEM((BE, HID), f32)] * NSL),
        *([pltpu.VMEM((BE, CH), f32)] * (3 * NSL)),
        *([pltpu.VMEM((2, BE), i32)] * NSL),
        *([pltpu.SemaphoreType.DMA] * (3 * NSL)),
    ],
)


def _dot(a, b):
    return jnp.dot(a, b, preferred_element_type=f32)


def _proj_body(x_ref, w_ref, b_ref, ws_ref, ssrc_ref, sdst_ref,
               xw_ref, as_ref, ad_ref):
    h = jnp.maximum(_dot(x_ref[...], w_ref[...]) + b_ref[...], 0.0)
    xw = _dot(h, ws_ref[...])
    xw_ref[...] = xw
    as_ref[...] = _dot(xw, ssrc_ref[...])
    ad_ref[...] = _dot(xw, sdst_ref[...])


def _proj(x, W_in, b_in2, Ws0, Ssrc0, Sdst0):
    return pl.pallas_call(
        _proj_body,
        grid=(N // RB,),
        in_specs=[
            pl.BlockSpec((RB, F_IN), lambda i: (i, 0)),
            pl.BlockSpec((F_IN, HID), lambda i: (0, 0)),
            pl.BlockSpec((1, HID), lambda i: (0, 0)),
            pl.BlockSpec((HID, HID), lambda i: (0, 0)),
            pl.BlockSpec((HID, CH), lambda i: (0, 0)),
            pl.BlockSpec((HID, CH), lambda i: (0, 0)),
        ],
        out_specs=[
            pl.BlockSpec((RB, HID), lambda i: (i, 0)),
            pl.BlockSpec((RB, CH), lambda i: (i, 0)),
            pl.BlockSpec((RB, CH), lambda i: (i, 0)),
        ],
        out_shape=[
            jax.ShapeDtypeStruct((N, HID), f32),
            jax.ShapeDtypeStruct((N, CH), f32),
            jax.ShapeDtypeStruct((N, CH), f32),
        ],
    )(x, W_in, b_in2, Ws0, Ssrc0, Sdst0)


def _layer_post(outp, esump, xw, a_s, a_d, sc, off, rmat):
    sv = a_s + a_d
    wself = jnp.exp(jnp.maximum(sv, sv * 0.2))
    tot = outp[0] + outp[1] + _dot(wself, rmat) * xw
    esum = esump[0] + esump[1] + wself
    recip = 1.0 / (esum + 1e-16)
    return jnp.maximum(tot * _dot(recip, rmat) * sc + off, 0.0)


def _combine_body(outp_ref, esump_ref, xw_ref, as_ref, ad_ref,
                  sc_ref, off_ref, rmat_ref, ws_ref, ssrc_ref, sdst_ref,
                  xwn_ref, asn_ref, adn_ref):
    hn = _layer_post(outp_ref[...], esump_ref[...], xw_ref[...],
                     as_ref[...], ad_ref[...],
                     sc_ref[...], off_ref[...], rmat_ref[...])
    xwn = _dot(hn, ws_ref[...])
    xwn_ref[...] = xwn
    asn_ref[...] = _dot(xwn, ssrc_ref[...])
    adn_ref[...] = _dot(xwn, sdst_ref[...])


def _combine(outp, esump, xw, As, Ad, sc, off, rmat, Wsn, Ssrcn, Sdstn):
    return pl.pallas_call(
        _combine_body,
        grid=(N // RB,),
        in_specs=[
            pl.BlockSpec((NC, RB, HID), lambda i: (0, i, 0)),
            pl.BlockSpec((NC, RB, CH), lambda i: (0, i, 0)),
            pl.BlockSpec((RB, HID), lambda i: (i, 0)),
            pl.BlockSpec((RB, CH), lambda i: (i, 0)),
            pl.BlockSpec((RB, CH), lambda i: (i, 0)),
            pl.BlockSpec((1, HID), lambda i: (0, 0)),
            pl.BlockSpec((1, HID), lambda i: (0, 0)),
            pl.BlockSpec((CH, HID), lambda i: (0, 0)),
            pl.BlockSpec((HID, HID), lambda i: (0, 0)),
            pl.BlockSpec((HID, CH), lambda i: (0, 0)),
            pl.BlockSpec((HID, CH), lambda i: (0, 0)),
        ],
        out_specs=[
            pl.BlockSpec((RB, HID), lambda i: (i, 0)),
            pl.BlockSpec((RB, CH), lambda i: (i, 0)),
            pl.BlockSpec((RB, CH), lambda i: (i, 0)),
        ],
        out_shape=[
            jax.ShapeDtypeStruct((N, HID), f32),
            jax.ShapeDtypeStruct((N, CH), f32),
            jax.ShapeDtypeStruct((N, CH), f32),
        ],
    )(outp, esump, xw, As, Ad, sc, off, rmat, Wsn, Ssrcn, Sdstn)


def _final_body(outp_ref, esump_ref, xw_ref, as_ref, ad_ref,
                sc_ref, off_ref, rmat_ref, batch_ref,
                w1_ref, b1_ref, w2_ref, b2_ref, out_ref,
                pool_acc, cnt_acc):
    i = pl.program_id(0)

    @pl.when(i == 0)
    def _():
        pool_acc[...] = jnp.zeros_like(pool_acc)
        cnt_acc[...] = jnp.zeros_like(cnt_acc)

    hn = _layer_post(outp_ref[...], esump_ref[...], xw_ref[...],
                     as_ref[...], ad_ref[...],
                     sc_ref[...], off_ref[...], rmat_ref[...])
    b2d = jnp.reshape(batch_ref[...], (1, RB))
    onehot_t = (lax.broadcasted_iota(i32, (NG, RB), 0) == b2d).astype(f32)
    pool_acc[...] += lax.dot_general(onehot_t, hn, (((1,), (0,)), ((), ())),
                                     preferred_element_type=f32)
    cnt_acc[...] += lax.dot_general(onehot_t, jnp.ones((RB, 1), f32),
                                    (((1,), (0,)), ((), ())),
                                    preferred_element_type=f32)

    @pl.when(i == (N // RB) - 1)
    def _():
        pooled = pool_acc[...] / jnp.maximum(cnt_acc[...], 1.0)
        t = jnp.maximum(_dot(pooled, w1_ref[...]) + b1_ref[...], 0.0)
        out_ref[...] = _dot(t, w2_ref[...]) + b2_ref[...]


def _final(outp, esump, xw, As, Ad, sc, off, rmat, batch3,
           W1cat, b1cat, W2bd, b2row):
    return pl.pallas_call(
        _final_body,
        grid=(N // RB,),
        in_specs=[
            pl.BlockSpec((NC, RB, HID), lambda i: (0, i, 0)),
            pl.BlockSpec((NC, RB, CH), lambda i: (0, i, 0)),
            pl.BlockSpec((RB, HID), lambda i: (i, 0)),
            pl.BlockSpec((RB, CH), lambda i: (i, 0)),
            pl.BlockSpec((RB, CH), lambda i: (i, 0)),
            pl.BlockSpec((1, HID), lambda i: (0, 0)),
            pl.BlockSpec((1, HID), lambda i: (0, 0)),
            pl.BlockSpec((CH, HID), lambda i: (0, 0)),
            pl.BlockSpec((1, 1, RB), lambda i: (i, 0, 0)),
            pl.BlockSpec((HID, 3 * NG), lambda i: (0, 0)),
            pl.BlockSpec((1, 3 * NG), lambda i: (0, 0)),
            pl.BlockSpec((3 * NG, 3), lambda i: (0, 0)),
            pl.BlockSpec((1, 3), lambda i: (0, 0)),
        ],
        out_specs=pl.BlockSpec((NG, 3), lambda i: (0, 0)),
        out_shape=jax.ShapeDtypeStruct((NG, 3), f32),
        scratch_shapes=[
            pltpu.VMEM((NG, HID), f32),
            pltpu.VMEM((NG, 1), f32),
        ],
    )(outp, esump, xw, As, Ad, sc, off, rmat, batch3,
      W1cat, b1cat, W2bd, b2row)


def kernel(x, edge_index, batch, W_in, b_in, Ws, att_src, att_dst, biases,
           bn_gamma, bn_beta, bn_mean, bn_var, HW1, Hb1, HW2, Hb2):
    eye8 = jnp.eye(NH, dtype=f32)
    S_src = (att_src[:, :, :, None] * eye8[:, None, :][None]).reshape(
        NL, HID, NH)
    S_src = jnp.pad(S_src, ((0, 0), (0, 0), (0, CH - NH)))
    S_dst = (att_dst[:, :, :, None] * eye8[:, None, :][None]).reshape(
        NL, HID, NH)
    S_dst = jnp.pad(S_dst, ((0, 0), (0, 0), (0, CH - NH)))

    scale = bn_gamma / jnp.sqrt(bn_var + 1e-5)
    off2 = biases * scale + (bn_beta - bn_mean * scale)

    rmat = (jnp.arange(HID, dtype=i32)[None, :] // CH
            == jnp.arange(CH, dtype=i32)[:, None]).astype(f32)

    W1cat = jnp.concatenate([HW1[0], HW1[1], HW1[2]], axis=1)
    b1cat = jnp.concatenate([Hb1[0], Hb1[1], Hb1[2]])[None, :]
    W2bd = (HW2[:, :, 0][:, :, None] * jnp.eye(3, dtype=f32)[:, None, :]
            ).reshape(3 * NG, 3)
    b2row = Hb2[:, 0][None, :]

    batch3 = batch.reshape(N // RB, 1, RB)
    b_in2 = b_in[None, :]

    xw, As, Ad = _proj(x, W_in, b_in2, Ws[0], S_src[0], S_dst[0])
    for l in range(NL):
        outp, esump = _sc_layer(xw, As, Ad, edge_index)
        if l + 1 < NL:
            xw, As, Ad = _combine(outp, esump, xw, As, Ad,
                                  scale[l][None], off2[l][None], rmat,
                                  Ws[l + 1], S_src[l + 1], S_dst[l + 1])
        else:
            out = _final(outp, esump, xw, As, Ad,
                         scale[l][None], off2[l][None], rmat, batch3,
                         W1cat, b1cat, W2bd, b2row)
    return out
